# baseline (speedup 1.0000x reference)
import jax
import jax.numpy as jnp
from jax.experimental import pallas as pl
from jax.experimental.pallas import tpu as pltpu

_INDEX = 2


def _copy_kernel(x_hbm, o_hbm, sem):
    copy = pltpu.make_async_copy(x_hbm.at[_INDEX], o_hbm, sem)
    copy.start()
    copy.wait()


def kernel(x):
    return pl.pallas_call(
        _copy_kernel,
        out_shape=jax.ShapeDtypeStruct(x.shape[1:], x.dtype),
        in_specs=[pl.BlockSpec(memory_space=pltpu.MemorySpace.HBM)],
        out_specs=pl.BlockSpec(memory_space=pltpu.MemorySpace.HBM),
        scratch_shapes=[pltpu.SemaphoreType.DMA],
    )(x)


# speedup vs baseline: 1.0023x; 1.0023x over previous
import jax
import jax.numpy as jnp
from jax.experimental import pallas as pl
from jax.experimental.pallas import tpu as pltpu

_INDEX = 2


_NCHUNK = 16


def _copy_kernel(x_hbm, o_hbm, sem):
    rows = o_hbm.shape[0]
    chunk = rows // _NCHUNK
    copies = []
    for i in range(_NCHUNK):
        c = pltpu.make_async_copy(
            x_hbm.at[_INDEX, pl.ds(i * chunk, chunk), :],
            o_hbm.at[pl.ds(i * chunk, chunk), :],
            sem.at[i],
        )
        c.start()
        copies.append(c)
    for c in copies:
        c.wait()


def kernel(x):
    return pl.pallas_call(
        _copy_kernel,
        out_shape=jax.ShapeDtypeStruct(x.shape[1:], x.dtype),
        in_specs=[pl.BlockSpec(memory_space=pltpu.MemorySpace.HBM)],
        out_specs=pl.BlockSpec(memory_space=pltpu.MemorySpace.HBM),
        scratch_shapes=[pltpu.SemaphoreType.DMA((_NCHUNK,))],
    )(x)


# speedup vs baseline: 49.0744x; 48.9599x over previous
import jax
import jax.numpy as jnp
from jax.experimental import pallas as pl
from jax.experimental.pallas import tpu as pltpu

_INDEX = 2


_BLOCK_ROWS = 512


def _copy_kernel(x_vmem, o_vmem):
    o_vmem[...] = x_vmem[0]


def kernel(x):
    _, rows, cols = x.shape
    grid = rows // _BLOCK_ROWS
    return pl.pallas_call(
        _copy_kernel,
        grid=(grid,),
        in_specs=[
            pl.BlockSpec((1, _BLOCK_ROWS, cols), lambda i: (_INDEX, i, 0))
        ],
        out_specs=pl.BlockSpec((_BLOCK_ROWS, cols), lambda i: (i, 0)),
        out_shape=jax.ShapeDtypeStruct(x.shape[1:], x.dtype),
    )(x)


# speedup vs baseline: 49.1540x; 1.0016x over previous
import jax
import jax.numpy as jnp
from jax.experimental import pallas as pl
from jax.experimental.pallas import tpu as pltpu

_INDEX = 2


_BLOCK_ROWS = 512


def _copy_kernel(x_vmem, o_vmem):
    o_vmem[...] = x_vmem[0]


def kernel(x):
    _, rows, cols = x.shape
    grid = rows // _BLOCK_ROWS
    return pl.pallas_call(
        _copy_kernel,
        grid=(grid,),
        in_specs=[
            pl.BlockSpec((1, _BLOCK_ROWS, cols), lambda i: (_INDEX, i, 0))
        ],
        out_specs=pl.BlockSpec((_BLOCK_ROWS, cols), lambda i: (i, 0)),
        out_shape=jax.ShapeDtypeStruct(x.shape[1:], x.dtype),
        compiler_params=pltpu.CompilerParams(
            dimension_semantics=("parallel",),
        ),
    )(x)
